# baseline (speedup 1.0000x reference)
import functools

import jax
import jax.numpy as jnp
from jax import lax
from jax.experimental import pallas as pl
from jax.experimental.pallas import tpu as pltpu
from jax.experimental.pallas import tpu_sc as plsc

_N = 10000
_E = 320000
_D = 128
_H = 4
_C = 32

_NPAD = 10240
_NWORKERS = 32
_CHUNK = _E // _NWORKERS
_LANES = 16

_BLK = 1000


def _sc_indicator_body(edge_hbm, out_hbm, idx_v, table_v):
    wid = lax.axis_index("s") * 2 + lax.axis_index("c")

    zeros = jnp.zeros((_LANES,), jnp.float32)

    def zero_body(i, _):
        for u in range(8):
            table_v[pl.ds(i * 8 * _LANES + u * _LANES, _LANES)] = zeros
        return 0

    lax.fori_loop(0, _NPAD // (8 * _LANES), zero_body, 0)

    pltpu.sync_copy(edge_hbm.at[pl.ds(_E + wid * _CHUNK, _CHUNK)], idx_v)

    ones = jnp.ones((_LANES,), jnp.float32)

    def scat_body(i, _):
        for u in range(5):
            idx = idx_v[pl.ds(i * 5 * _LANES + u * _LANES, _LANES)]
            plsc.store_scatter(table_v, [idx], ones)
        return 0

    lax.fori_loop(0, _CHUNK // (5 * _LANES), scat_body, 0)

    pltpu.sync_copy(table_v, out_hbm.at[wid])


_sc_indicator = pl.kernel(
    _sc_indicator_body,
    out_type=jax.ShapeDtypeStruct((_NWORKERS, _NPAD), jnp.float32),
    mesh=plsc.VectorSubcoreMesh(core_axis_name="c", subcore_axis_name="s"),
    compiler_params=pltpu.CompilerParams(needs_layout_passes=False),
    scratch_types=[
        pltpu.VMEM((_CHUNK,), jnp.int32),
        pltpu.VMEM((_NPAD,), jnp.float32),
    ],
)


def _tc_proj_body(x_ref, wr_ref, br_ref, out_ref):
    out_ref[...] = jnp.dot(
        x_ref[...], wr_ref[...],
        preferred_element_type=jnp.float32) + br_ref[...]


_tc_proj = pl.pallas_call(
    _tc_proj_body,
    grid=(_N // _BLK,),
    in_specs=[
        pl.BlockSpec((_BLK, _D), lambda i: (i, 0)),
        pl.BlockSpec((_D, _D), lambda i: (0, 0)),
        pl.BlockSpec((1, _D), lambda i: (0, 0)),
    ],
    out_specs=pl.BlockSpec((_BLK, _D), lambda i: (i, 0)),
    out_shape=jax.ShapeDtypeStruct((_N, _D), jnp.float32),
)


def _tc_fused_body(k_ref, rar_ref, g_ref, b_ref, cnt_ref,
                   blk_ref, out_ref):
    k = k_ref[...]
    s = jnp.sum(cnt_ref[...], axis=1, keepdims=True)
    mask = (s > 0.0).astype(jnp.float32)
    agg = k * mask
    rar = rar_ref[...]
    d = jax.nn.relu(agg * rar) - jax.nn.relu(k * rar)
    w0 = jax.nn.sigmoid(
        jnp.dot(d, blk_ref[...], preferred_element_type=jnp.float32))
    out = agg * w0 + k * (1.0 - w0)
    out = jax.nn.relu(out)
    m = jnp.mean(out, axis=1, keepdims=True)
    c = out - m
    v = jnp.mean(c * c, axis=1, keepdims=True)
    out_ref[...] = c * lax.rsqrt(v + 1e-5) * g_ref[...] + b_ref[...]


_tc_fused = pl.pallas_call(
    _tc_fused_body,
    grid=(_N // _BLK,),
    in_specs=[
        pl.BlockSpec((_BLK, _D), lambda i: (i, 0)),
        pl.BlockSpec((1, _D), lambda i: (0, 0)),
        pl.BlockSpec((1, _D), lambda i: (0, 0)),
        pl.BlockSpec((1, _D), lambda i: (0, 0)),
        pl.BlockSpec((_BLK, _NWORKERS), lambda i: (i, 0)),
        pl.BlockSpec((_D, _D), lambda i: (0, 0)),
    ],
    out_specs=pl.BlockSpec((_BLK, _D), lambda i: (i, 0)),
    out_shape=jax.ShapeDtypeStruct((_N, _D), jnp.float32),
)


def kernel(x, edge_index, Wl, bl, Wr, br, attn_l, attn_r, rel_attn_l,
           rel_attn_r, ln_gamma, ln_beta):
    del Wl, bl, attn_l, attn_r, rel_attn_l
    tables = _sc_indicator(edge_index.reshape(-1))
    cnt_t = tables.T

    k = _tc_proj(x, Wr, br.reshape(1, _D))

    hid = jnp.arange(_D, dtype=jnp.int32) // _C
    blk = (hid[:, None] == hid[None, :]).astype(jnp.float32)
    return _tc_fused(
        k,
        rel_attn_r.reshape(1, _D),
        ln_gamma.reshape(1, _D),
        ln_beta.reshape(1, _D),
        cnt_t,
        blk,
    )


# speedup vs baseline: 1.1103x; 1.1103x over previous
import functools

import jax
import jax.numpy as jnp
from jax import lax
from jax.experimental import pallas as pl
from jax.experimental.pallas import tpu as pltpu
from jax.experimental.pallas import tpu_sc as plsc

_N = 10000
_E = 320000
_D = 128
_H = 4
_C = 32

_NPAD = 10240
_NWORKERS = 32
_CHUNK = _E // _NWORKERS
_LANES = 16

_BLK = 1000


def _sc_indicator_body(edge_hbm, out_hbm, idx_v, table_v):
    wid = lax.axis_index("s") * 2 + lax.axis_index("c")

    def run(sem):
        cp = pltpu.make_async_copy(
            edge_hbm.at[pl.ds(_E + wid * _CHUNK, _CHUNK)], idx_v, sem)
        cp.start()

        zeros = jnp.zeros((_LANES,), jnp.float32)

        def zero_body(i, _):
            for u in range(8):
                table_v[pl.ds(i * 8 * _LANES + u * _LANES, _LANES)] = zeros
            return 0

        lax.fori_loop(0, _NPAD // (8 * _LANES), zero_body, 0)
        cp.wait()

    pl.run_scoped(run, pltpu.SemaphoreType.DMA)

    ones = jnp.ones((_LANES,), jnp.float32)

    def scat_body(i, _):
        for u in range(25):
            idx = idx_v[pl.ds(i * 25 * _LANES + u * _LANES, _LANES)]
            plsc.store_scatter(table_v, [idx], ones)
        return 0

    lax.fori_loop(0, _CHUNK // (25 * _LANES), scat_body, 0)

    pltpu.sync_copy(table_v, out_hbm.at[wid])


_sc_indicator = pl.kernel(
    _sc_indicator_body,
    out_type=jax.ShapeDtypeStruct((_NWORKERS, _NPAD), jnp.float32),
    mesh=plsc.VectorSubcoreMesh(core_axis_name="c", subcore_axis_name="s"),
    compiler_params=pltpu.CompilerParams(needs_layout_passes=False),
    scratch_types=[
        pltpu.VMEM((_CHUNK,), jnp.int32),
        pltpu.VMEM((_NPAD,), jnp.float32),
    ],
)


def _tc_fused_body(x_ref, wr_ref, br_ref, rar_ref, g_ref, b_ref, cnt_ref,
                   blk_ref, out_ref):
    k = jnp.dot(x_ref[...], wr_ref[...], preferred_element_type=jnp.float32)
    k = k + br_ref[...]
    s = jnp.sum(cnt_ref[...], axis=1, keepdims=True)
    u = (s <= 0.0).astype(jnp.float32)
    t = jax.nn.relu(k * rar_ref[...]) * u
    w0 = jax.nn.sigmoid(
        -jnp.dot(t, blk_ref[...], preferred_element_type=jnp.float32))
    out = jax.nn.relu(k * (1.0 - w0 * u))
    m = jnp.mean(out, axis=1, keepdims=True)
    c = out - m
    v = jnp.mean(c * c, axis=1, keepdims=True)
    out_ref[...] = c * lax.rsqrt(v + 1e-5) * g_ref[...] + b_ref[...]


_tc_fused = pl.pallas_call(
    _tc_fused_body,
    grid=(_N // _BLK,),
    in_specs=[
        pl.BlockSpec((_BLK, _D), lambda i: (i, 0)),
        pl.BlockSpec((_D, _D), lambda i: (0, 0)),
        pl.BlockSpec((1, _D), lambda i: (0, 0)),
        pl.BlockSpec((1, _D), lambda i: (0, 0)),
        pl.BlockSpec((1, _D), lambda i: (0, 0)),
        pl.BlockSpec((1, _D), lambda i: (0, 0)),
        pl.BlockSpec((_BLK, _NWORKERS), lambda i: (i, 0)),
        pl.BlockSpec((_D, _D), lambda i: (0, 0)),
    ],
    out_specs=pl.BlockSpec((_BLK, _D), lambda i: (i, 0)),
    out_shape=jax.ShapeDtypeStruct((_N, _D), jnp.float32),
)


def kernel(x, edge_index, Wl, bl, Wr, br, attn_l, attn_r, rel_attn_l,
           rel_attn_r, ln_gamma, ln_beta):
    del Wl, bl, attn_l, attn_r, rel_attn_l
    tables = _sc_indicator(edge_index.reshape(-1))
    cnt_t = tables.T

    hid = jnp.arange(_D, dtype=jnp.int32) // _C
    blk = (hid[:, None] == hid[None, :]).astype(jnp.float32)
    return _tc_fused(
        x, Wr,
        br.reshape(1, _D),
        rel_attn_r.reshape(1, _D),
        ln_gamma.reshape(1, _D),
        ln_beta.reshape(1, _D),
        cnt_t,
        blk,
    )


# speedup vs baseline: 1.1790x; 1.0619x over previous
import functools

import jax
import jax.numpy as jnp
from jax import lax
from jax.experimental import pallas as pl
from jax.experimental.pallas import tpu as pltpu
from jax.experimental.pallas import tpu_sc as plsc

_N = 10000
_E = 320000
_D = 128
_H = 4
_C = 32

_NPAD = 10240
_NWORKERS = 32
_CHUNK = _E // _NWORKERS
_LANES = 16

_BLK = 2000


def _sc_indicator_body(edge_hbm, out_hbm, idx_v, table_v):
    wid = lax.axis_index("s") * 2 + lax.axis_index("c")

    def run(sem):
        cp = pltpu.make_async_copy(
            edge_hbm.at[pl.ds(_E + wid * _CHUNK, _CHUNK)], idx_v, sem)
        cp.start()

        zeros = jnp.zeros((_LANES,), jnp.float32)

        def zero_body(i, _):
            for u in range(8):
                table_v[pl.ds(i * 8 * _LANES + u * _LANES, _LANES)] = zeros
            return 0

        lax.fori_loop(0, _NPAD // (8 * _LANES), zero_body, 0)
        cp.wait()

    pl.run_scoped(run, pltpu.SemaphoreType.DMA)

    ones = jnp.ones((_LANES,), jnp.float32)

    def scat_body(i, _):
        for u in range(25):
            idx = idx_v[pl.ds(i * 25 * _LANES + u * _LANES, _LANES)]
            plsc.store_scatter(table_v, [idx], ones)
        return 0

    lax.fori_loop(0, _CHUNK // (25 * _LANES), scat_body, 0)

    pltpu.sync_copy(table_v, out_hbm.at[wid])


_sc_indicator = pl.kernel(
    _sc_indicator_body,
    out_type=jax.ShapeDtypeStruct((_NWORKERS, _NPAD), jnp.float32),
    mesh=plsc.VectorSubcoreMesh(core_axis_name="c", subcore_axis_name="s"),
    compiler_params=pltpu.CompilerParams(needs_layout_passes=False),
    scratch_types=[
        pltpu.VMEM((_CHUNK,), jnp.int32),
        pltpu.VMEM((_NPAD,), jnp.float32),
    ],
)


def _tc_fused_body(x_ref, wr_ref, br_ref, rar_ref, g_ref, b_ref, cnt_ref,
                   blk_ref, out_ref):
    k = jnp.dot(x_ref[...], wr_ref[...], preferred_element_type=jnp.float32)
    k = k + br_ref[...]
    s = jnp.sum(cnt_ref[...], axis=1, keepdims=True)
    u = (s <= 0.0).astype(jnp.float32)
    t = jax.nn.relu(k * rar_ref[...]) * u
    w0 = jax.nn.sigmoid(
        -jnp.dot(t, blk_ref[...], preferred_element_type=jnp.float32))
    out = jax.nn.relu(k * (1.0 - w0 * u))
    m = jnp.mean(out, axis=1, keepdims=True)
    c = out - m
    v = jnp.mean(c * c, axis=1, keepdims=True)
    out_ref[...] = c * lax.rsqrt(v + 1e-5) * g_ref[...] + b_ref[...]


_tc_fused = pl.pallas_call(
    _tc_fused_body,
    grid=(_N // _BLK,),
    in_specs=[
        pl.BlockSpec((_BLK, _D), lambda i: (i, 0)),
        pl.BlockSpec((_D, _D), lambda i: (0, 0)),
        pl.BlockSpec((1, _D), lambda i: (0, 0)),
        pl.BlockSpec((1, _D), lambda i: (0, 0)),
        pl.BlockSpec((1, _D), lambda i: (0, 0)),
        pl.BlockSpec((1, _D), lambda i: (0, 0)),
        pl.BlockSpec((_BLK, _NWORKERS), lambda i: (i, 0)),
        pl.BlockSpec((_D, _D), lambda i: (0, 0)),
    ],
    out_specs=pl.BlockSpec((_BLK, _D), lambda i: (i, 0)),
    out_shape=jax.ShapeDtypeStruct((_N, _D), jnp.float32),
)


def kernel(x, edge_index, Wl, bl, Wr, br, attn_l, attn_r, rel_attn_l,
           rel_attn_r, ln_gamma, ln_beta):
    del Wl, bl, attn_l, attn_r, rel_attn_l
    tables = _sc_indicator(edge_index.reshape(-1))
    cnt_t = tables.T

    hid = jnp.arange(_D, dtype=jnp.int32) // _C
    blk = (hid[:, None] == hid[None, :]).astype(jnp.float32)
    return _tc_fused(
        x, Wr,
        br.reshape(1, _D),
        rel_attn_r.reshape(1, _D),
        ln_gamma.reshape(1, _D),
        ln_beta.reshape(1, _D),
        cnt_t,
        blk,
    )


# speedup vs baseline: 1.2663x; 1.0741x over previous
import functools

import jax
import jax.numpy as jnp
from jax import lax
from jax.experimental import pallas as pl
from jax.experimental.pallas import tpu as pltpu
from jax.experimental.pallas import tpu_sc as plsc

_N = 10000
_E = 320000
_D = 128
_H = 4
_C = 32

_NPAD = 10240
_NWORKERS = 32
_CHUNK = _E // _NWORKERS
_LANES = 16

_BLK = 2000


def _sc_indicator_body(edge_hbm, out_hbm, idx_v, table_v):
    wid = lax.axis_index("s") * 2 + lax.axis_index("c")

    def run(sem):
        cp = pltpu.make_async_copy(
            edge_hbm.at[pl.ds(_E + wid * _CHUNK, _CHUNK)], idx_v, sem)
        cp.start()

        zeros = jnp.zeros((_LANES,), jnp.float32)

        @plsc.parallel_loop(0, _NPAD // _LANES, unroll=8)
        def zero_body(i):
            table_v[pl.ds(i * _LANES, _LANES)] = zeros

        cp.wait()

    pl.run_scoped(run, pltpu.SemaphoreType.DMA)

    ones = jnp.ones((_LANES,), jnp.float32)

    @plsc.parallel_loop(0, _CHUNK // _LANES, unroll=8)
    def scat_body(i):
        idx = idx_v[pl.ds(i * _LANES, _LANES)]
        plsc.store_scatter(table_v, [idx], ones)

    pltpu.sync_copy(table_v, out_hbm.at[wid])


_sc_indicator = pl.kernel(
    _sc_indicator_body,
    out_type=jax.ShapeDtypeStruct((_NWORKERS, _NPAD), jnp.float32),
    mesh=plsc.VectorSubcoreMesh(core_axis_name="c", subcore_axis_name="s"),
    compiler_params=pltpu.CompilerParams(needs_layout_passes=False),
    scratch_types=[
        pltpu.VMEM((_CHUNK,), jnp.int32),
        pltpu.VMEM((_NPAD,), jnp.float32),
    ],
)


def _tc_fused_body(x_ref, wr_ref, br_ref, rar_ref, g_ref, b_ref, cnt_ref,
                   blk_ref, out_ref):
    k = jnp.dot(x_ref[...], wr_ref[...], preferred_element_type=jnp.float32)
    k = k + br_ref[...]
    s = jnp.sum(cnt_ref[...], axis=1, keepdims=True)
    u = (s <= 0.0).astype(jnp.float32)
    t = jax.nn.relu(k * rar_ref[...]) * u
    w0 = jax.nn.sigmoid(
        -jnp.dot(t, blk_ref[...], preferred_element_type=jnp.float32))
    out = jax.nn.relu(k * (1.0 - w0 * u))
    m = jnp.mean(out, axis=1, keepdims=True)
    c = out - m
    v = jnp.mean(c * c, axis=1, keepdims=True)
    out_ref[...] = c * lax.rsqrt(v + 1e-5) * g_ref[...] + b_ref[...]


_tc_fused = pl.pallas_call(
    _tc_fused_body,
    grid=(_N // _BLK,),
    in_specs=[
        pl.BlockSpec((_BLK, _D), lambda i: (i, 0)),
        pl.BlockSpec((_D, _D), lambda i: (0, 0)),
        pl.BlockSpec((1, _D), lambda i: (0, 0)),
        pl.BlockSpec((1, _D), lambda i: (0, 0)),
        pl.BlockSpec((1, _D), lambda i: (0, 0)),
        pl.BlockSpec((1, _D), lambda i: (0, 0)),
        pl.BlockSpec((_BLK, _NWORKERS), lambda i: (i, 0)),
        pl.BlockSpec((_D, _D), lambda i: (0, 0)),
    ],
    out_specs=pl.BlockSpec((_BLK, _D), lambda i: (i, 0)),
    out_shape=jax.ShapeDtypeStruct((_N, _D), jnp.float32),
)


def kernel(x, edge_index, Wl, bl, Wr, br, attn_l, attn_r, rel_attn_l,
           rel_attn_r, ln_gamma, ln_beta):
    del Wl, bl, attn_l, attn_r, rel_attn_l
    tables = _sc_indicator(edge_index.reshape(-1))
    cnt_t = tables.T

    hid = jnp.arange(_D, dtype=jnp.int32) // _C
    blk = (hid[:, None] == hid[None, :]).astype(jnp.float32)
    return _tc_fused(
        x, Wr,
        br.reshape(1, _D),
        rel_attn_r.reshape(1, _D),
        ln_gamma.reshape(1, _D),
        ln_beta.reshape(1, _D),
        cnt_t,
        blk,
    )


# speedup vs baseline: 1.2951x; 1.0227x over previous
import functools

import jax
import jax.numpy as jnp
from jax import lax
from jax.experimental import pallas as pl
from jax.experimental.pallas import tpu as pltpu
from jax.experimental.pallas import tpu_sc as plsc

_N = 10000
_E = 320000
_D = 128
_H = 4
_C = 32

_NPAD = 10240
_NWORKERS = 32
_CHUNK = _E // _NWORKERS
_LANES = 16

_BLK = 5000


def _sc_indicator_body(edge_hbm, out_hbm, idx_v, table_v):
    wid = lax.axis_index("s") * 2 + lax.axis_index("c")

    def run(sem):
        cp = pltpu.make_async_copy(
            edge_hbm.at[pl.ds(_E + wid * _CHUNK, _CHUNK)], idx_v, sem)
        cp.start()

        zeros = jnp.zeros((_LANES,), jnp.float32)

        @plsc.parallel_loop(0, _NPAD // _LANES, unroll=8)
        def zero_body(i):
            table_v[pl.ds(i * _LANES, _LANES)] = zeros

        cp.wait()

    pl.run_scoped(run, pltpu.SemaphoreType.DMA)

    ones = jnp.ones((_LANES,), jnp.float32)

    @plsc.parallel_loop(0, _CHUNK // _LANES, unroll=8)
    def scat_body(i):
        idx = idx_v[pl.ds(i * _LANES, _LANES)]
        plsc.store_scatter(table_v, [idx], ones)

    pltpu.sync_copy(table_v, out_hbm.at[wid])


_sc_indicator = pl.kernel(
    _sc_indicator_body,
    out_type=jax.ShapeDtypeStruct((_NWORKERS, _NPAD), jnp.float32),
    mesh=plsc.VectorSubcoreMesh(core_axis_name="c", subcore_axis_name="s"),
    compiler_params=pltpu.CompilerParams(needs_layout_passes=False),
    scratch_types=[
        pltpu.VMEM((_CHUNK,), jnp.int32),
        pltpu.VMEM((_NPAD,), jnp.float32),
    ],
)


def _tc_fused_body(x_ref, wr_ref, br_ref, rar_ref, g_ref, b_ref, cnt_ref,
                   blk_ref, out_ref):
    k = jnp.dot(x_ref[...], wr_ref[...], preferred_element_type=jnp.float32)
    k = k + br_ref[...]
    s = jnp.sum(cnt_ref[...], axis=1, keepdims=True)
    u = (s <= 0.0).astype(jnp.float32)
    t = jax.nn.relu(k * rar_ref[...]) * u
    w0 = jax.nn.sigmoid(
        -jnp.dot(t, blk_ref[...], preferred_element_type=jnp.float32))
    out = jax.nn.relu(k * (1.0 - w0 * u))
    m = jnp.mean(out, axis=1, keepdims=True)
    c = out - m
    v = jnp.mean(c * c, axis=1, keepdims=True)
    out_ref[...] = c * lax.rsqrt(v + 1e-5) * g_ref[...] + b_ref[...]


_tc_fused = pl.pallas_call(
    _tc_fused_body,
    grid=(_N // _BLK,),
    in_specs=[
        pl.BlockSpec((_BLK, _D), lambda i: (i, 0)),
        pl.BlockSpec((_D, _D), lambda i: (0, 0)),
        pl.BlockSpec((1, _D), lambda i: (0, 0)),
        pl.BlockSpec((1, _D), lambda i: (0, 0)),
        pl.BlockSpec((1, _D), lambda i: (0, 0)),
        pl.BlockSpec((1, _D), lambda i: (0, 0)),
        pl.BlockSpec((_BLK, _NWORKERS), lambda i: (i, 0)),
        pl.BlockSpec((_D, _D), lambda i: (0, 0)),
    ],
    out_specs=pl.BlockSpec((_BLK, _D), lambda i: (i, 0)),
    out_shape=jax.ShapeDtypeStruct((_N, _D), jnp.float32),
)


def kernel(x, edge_index, Wl, bl, Wr, br, attn_l, attn_r, rel_attn_l,
           rel_attn_r, ln_gamma, ln_beta):
    del Wl, bl, attn_l, attn_r, rel_attn_l
    tables = _sc_indicator(edge_index.reshape(-1))
    cnt_t = tables.T

    hid = jnp.arange(_D, dtype=jnp.int32) // _C
    blk = (hid[:, None] == hid[None, :]).astype(jnp.float32)
    return _tc_fused(
        x, Wr,
        br.reshape(1, _D),
        rel_attn_r.reshape(1, _D),
        ln_gamma.reshape(1, _D),
        ln_beta.reshape(1, _D),
        cnt_t,
        blk,
    )
